# baseline (speedup 1.0000x reference)
import functools

import jax
import jax.numpy as jnp
from jax import lax
from jax.experimental import pallas as pl
from jax.experimental.pallas import tpu as pltpu
from jax.experimental.pallas import tpu_sc as plsc

N_NODES = 10000
ACC_ROWS = 10008
D_FEAT = 128
HIDDEN = 256
N_CLASSES = 40
N_EDGES = 320000

NC = 2
NS = 16
NW = NC * NS
LANES = 16

E_PER_W = N_EDGES // NW
CHUNK = 128
SUB = 8
SUP_A = 10
SUP_B = 20
HALF = HIDDEN // 2
OWN = 624


@functools.partial(
    pl.kernel,
    out_type=jax.ShapeDtypeStruct((NW, N_NODES), jnp.float32),
    mesh=plsc.VectorSubcoreMesh(core_axis_name="c", subcore_axis_name="s"),
    scratch_types=[
        pltpu.VMEM((E_PER_W,), jnp.int32),
        pltpu.VMEM((N_NODES,), jnp.float32),
    ],
    compiler_params=pltpu.CompilerParams(needs_layout_passes=False),
)
def _deg_kernel(dst_hbm, out_hbm, idx_v, deg_v):
    c = lax.axis_index("c")
    s = lax.axis_index("s")
    wid = s * NC + c
    pltpu.sync_copy(dst_hbm.at[pl.ds(wid * E_PER_W, E_PER_W)], idx_v)

    zeros = jnp.zeros((LANES,), jnp.float32)

    @pl.loop(0, N_NODES // LANES)
    def _(i):
        deg_v[pl.ds(i * LANES, LANES)] = zeros

    ones = jnp.ones((LANES,), jnp.float32)

    @pl.loop(0, E_PER_W // LANES)
    def _(i):
        idx = idx_v[pl.ds(i * LANES, LANES)]
        plsc.addupdate_scatter(deg_v, [idx], ones)

    pltpu.sync_copy(deg_v, out_hbm.at[wid])


def _make_scatter_kernel(sup, shared_table=False):
    @functools.partial(
        pl.kernel,
        out_type=jax.ShapeDtypeStruct((NC, N_NODES, HALF), jnp.float32),
        mesh=plsc.VectorSubcoreMesh(core_axis_name="c", subcore_axis_name="s"),
        scratch_types=[
            pltpu.VMEM((SUB, CHUNK), jnp.int32),
            pltpu.VMEM((SUB, CHUNK), jnp.int32),
            pltpu.VMEM((SUB, CHUNK), jnp.int32),
            pltpu.VMEM((SUB, CHUNK), jnp.int32),
            pltpu.VMEM((CHUNK, HALF), jnp.float32),
            pltpu.VMEM((CHUNK, HALF), jnp.float32),
            pltpu.VMEM_SHARED((ACC_ROWS, HALF), jnp.float32),
            pltpu.SemaphoreType.DMA,
            pltpu.SemaphoreType.DMA,
            pltpu.SemaphoreType.DMA,
            pltpu.SemaphoreType.DMA,
            pltpu.SemaphoreType.DMA,
        ],
    )
    def _scatter_kernel(g_hbm, src_hbm, dst_hbm, out_hbm,
                        isrc_a, isrc_b, idst_a, idst_b, buf_a, buf_b, acc,
                        sem_ga, sem_gb, sem_sa, sem_sb, sem_i):
        c = lax.axis_index("c")
        s = lax.axis_index("s")
        isrc = (isrc_a, isrc_b)
        idst = (idst_a, idst_b)
        bufs = (buf_a, buf_b)
        gsems = (sem_ga, sem_gb)
        ssems = (sem_sa, sem_sb)

        pltpu.sync_copy(src_hbm.at[c, s, pl.ds(0, SUB)], isrc_a)
        pltpu.sync_copy(dst_hbm.at[c, s, pl.ds(0, SUB)], idst_a)

        zeros = jnp.zeros((LANES,), jnp.float32)

        @pl.loop(0, CHUNK)
        def _(i):
            @pl.loop(0, HALF // LANES)
            def _(k):
                buf_a[i, pl.ds(k * LANES, LANES)] = zeros

        zrows = 104
        zbase = pl.multiple_of(s * OWN, 8)

        @pl.loop(0, OWN // zrows)
        def _(i):
            pltpu.sync_copy(buf_a.at[pl.ds(0, zrows)],
                            acc.at[pl.ds(zbase + i * zrows, zrows)])

        @pl.when(s == NS - 1)
        def _():
            pltpu.sync_copy(buf_a.at[pl.ds(0, 16)],
                            acc.at[pl.ds(NS * OWN, 16)])

        plsc.subcore_barrier()

        g_c = g_hbm if shared_table else g_hbm.at[c]
        pltpu.async_copy(g_c.at[isrc_a.at[0]], buf_a, sem_ga)

        def super_step(m, sel):
            isrc_cur, idst_cur = isrc[sel], idst[sel]
            isrc_nxt, idst_nxt = isrc[1 - sel], idst[1 - sel]

            @pl.when(m < sup - 1)
            def _():
                nxt = pl.ds((m + 1) * SUB, SUB)
                pltpu.async_copy(src_hbm.at[c, s, nxt], isrc_nxt, sem_i)
                pltpu.async_copy(dst_hbm.at[c, s, nxt], idst_nxt, sem_i)

            for k in range(SUB):
                x = k % 2
                buf, other = bufs[x], bufs[1 - x]
                pltpu.make_async_copy(g_c.at[isrc_cur.at[k]], buf,
                                      gsems[x]).wait()
                pltpu.async_copy(buf, acc.at[idst_cur.at[k]], ssems[x],
                                 add=True)

                def wait_prev_scatter():
                    pltpu.make_async_copy(other, acc.at[idst_cur.at[k]],
                                          ssems[1 - x]).wait()

                if k > 0:
                    wait_prev_scatter()
                else:
                    @pl.when(m > 0)
                    def _():
                        wait_prev_scatter()

                if k < SUB - 1:
                    pltpu.async_copy(g_c.at[isrc_cur.at[k + 1]], other,
                                     gsems[1 - x])
                else:
                    @pl.when(m < sup - 1)
                    def _():
                        nxt = pl.ds((m + 1) * SUB, SUB)
                        pltpu.make_async_copy(
                            src_hbm.at[c, s, nxt], isrc_nxt, sem_i).wait()
                        pltpu.make_async_copy(
                            dst_hbm.at[c, s, nxt], idst_nxt, sem_i).wait()
                        pltpu.async_copy(g_c.at[isrc_nxt.at[0]], other,
                                         gsems[1 - x])

        @pl.loop(0, sup // 2)
        def _(mm):
            super_step(mm * 2, 0)
            super_step(mm * 2 + 1, 1)

        pltpu.make_async_copy(bufs[1], acc.at[idst_b.at[SUB - 1]],
                              ssems[1]).wait()

        plsc.subcore_barrier()
        wbase = pl.multiple_of(s * OWN, 8)
        pltpu.sync_copy(acc.at[pl.ds(wbase, OWN)],
                        out_hbm.at[c, pl.ds(wbase, OWN)])

        @pl.when(s == NS - 1)
        def _():
            pltpu.sync_copy(acc.at[pl.ds(NS * OWN, 16)],
                            out_hbm.at[c, pl.ds(NS * OWN, 16)])

    return _scatter_kernel


_scatter_a = _make_scatter_kernel(SUP_A, shared_table=True)
_scatter_b = _make_scatter_kernel(SUP_B)


_RB = 2000
_GRID = N_NODES // _RB


def _tc1_body(x_ref, dp_ref, y_ref, dinv_ref):
    deg = jnp.sum(dp_ref[...], axis=1, keepdims=True) + 1.0
    dinv = lax.rsqrt(deg)
    y_ref[...] = x_ref[...] * dinv
    dinv_ref[...] = dinv


def _tc1(x, dp):
    return pl.pallas_call(
        _tc1_body,
        grid=(_GRID,),
        in_specs=[
            pl.BlockSpec((_RB, D_FEAT), lambda i: (i, 0)),
            pl.BlockSpec((_RB, NW), lambda i: (i, 0)),
        ],
        out_specs=[
            pl.BlockSpec((_RB, D_FEAT), lambda i: (i, 0)),
            pl.BlockSpec((_RB, 1), lambda i: (i, 0)),
        ],
        out_shape=[
            jax.ShapeDtypeStruct((N_NODES, D_FEAT), jnp.float32),
            jax.ShapeDtypeStruct((N_NODES, 1), jnp.float32),
        ],
    )(x, dp)


def _tc2_body(s_ref, y_ref, dinv_ref, b_ref, w_ref, y2_ref):
    dinv = dinv_ref[...]
    z = s_ref[0] + s_ref[1] + y_ref[...]
    h = jnp.maximum(
        dinv * jnp.dot(z, w_ref[...], preferred_element_type=jnp.float32)
        + b_ref[...], 0.0)
    y2 = h * dinv
    y2_ref[0] = y2[:, :HALF]
    y2_ref[1] = y2[:, HALF:]


def _tc2(Sy1, y1, dinv, b1, W1):
    return pl.pallas_call(
        _tc2_body,
        grid=(_GRID,),
        in_specs=[
            pl.BlockSpec((NC, _RB, D_FEAT), lambda i: (0, i, 0)),
            pl.BlockSpec((_RB, D_FEAT), lambda i: (i, 0)),
            pl.BlockSpec((_RB, 1), lambda i: (i, 0)),
            pl.BlockSpec((1, HIDDEN), lambda i: (0, 0)),
            pl.BlockSpec((D_FEAT, HIDDEN), lambda i: (0, 0)),
        ],
        out_specs=pl.BlockSpec((NC, _RB, HALF), lambda i: (0, i, 0)),
        out_shape=jax.ShapeDtypeStruct((NC, N_NODES, HALF), jnp.float32),
    )(Sy1, y1, dinv, b1, W1)


def _tc3_body(s_ref, y2_ref, dinv_ref, b_ref, w_ref, wo_ref, bo_ref,
              out_ref):
    dinv = dinv_ref[...]
    w = w_ref[...]
    z0 = s_ref[0] + y2_ref[0]
    z1 = s_ref[1] + y2_ref[1]
    h = jnp.maximum(
        dinv * (jnp.dot(z0, w[:HALF], preferred_element_type=jnp.float32)
                + jnp.dot(z1, w[HALF:], preferred_element_type=jnp.float32))
        + b_ref[...], 0.0)
    logits = (jnp.dot(h, wo_ref[...], preferred_element_type=jnp.float32)
              + bo_ref[...])
    m = jnp.max(logits, axis=1, keepdims=True)
    e = jnp.exp(logits - m)
    out_ref[...] = e / jnp.sum(e, axis=1, keepdims=True)


def _tc3(S2, y2, dinv, b2, W2, Wout, bout):
    return pl.pallas_call(
        _tc3_body,
        grid=(_GRID,),
        in_specs=[
            pl.BlockSpec((NC, _RB, HALF), lambda i: (0, i, 0)),
            pl.BlockSpec((NC, _RB, HALF), lambda i: (0, i, 0)),
            pl.BlockSpec((_RB, 1), lambda i: (i, 0)),
            pl.BlockSpec((1, HIDDEN), lambda i: (0, 0)),
            pl.BlockSpec((HIDDEN, HIDDEN), lambda i: (0, 0)),
            pl.BlockSpec((HIDDEN, N_CLASSES), lambda i: (0, 0)),
            pl.BlockSpec((1, N_CLASSES), lambda i: (0, 0)),
        ],
        out_specs=pl.BlockSpec((_RB, N_CLASSES), lambda i: (i, 0)),
        out_shape=jax.ShapeDtypeStruct((N_NODES, N_CLASSES), jnp.float32),
    )(S2, y2, dinv, b2, W2, Wout, bout)


def _pad_edges_split(idx, fill):
    n_real = N_EDGES // (NC * NS)
    n_pad = SUP_A * SUB * CHUNK
    idx3 = idx.reshape(NC, NS, n_real)
    pad = jnp.full((NC, NS, n_pad - n_real), fill, jnp.int32)
    return jnp.concatenate([idx3, pad], axis=2).reshape(
        NC, NS, SUP_A * SUB, CHUNK)


def _pad_edges_full(idx, fill):
    n_real = N_EDGES // NS
    n_pad = SUP_B * SUB * CHUNK
    idx2 = idx.reshape(NS, n_real)
    pad = jnp.full((NS, n_pad - n_real), fill, jnp.int32)
    arr = jnp.concatenate([idx2, pad], axis=1).reshape(
        NS, SUP_B * SUB, CHUNK)
    return jnp.broadcast_to(arr[None], (NC,) + arr.shape)


@jax.jit
def kernel(x, edge_index, W1, b1, W2, b2, Wout, bout):
    src = edge_index[0].astype(jnp.int32)
    dst = edge_index[1].astype(jnp.int32)

    deg_parts = _deg_kernel(dst)
    dp = deg_parts.T

    srcA = _pad_edges_split(src, 0)
    dstA = _pad_edges_split(dst, N_NODES)
    srcB = _pad_edges_full(src, 0)
    dstB = _pad_edges_full(dst, N_NODES)

    y1, dinv = _tc1(x, dp)
    Sy1 = _scatter_a(y1, srcA, dstA)
    y2 = _tc2(Sy1, y1, dinv, b1.reshape(1, HIDDEN), W1)
    S2 = _scatter_b(y2, srcB, dstB)
    return _tc3(S2, y2, dinv, b2.reshape(1, HIDDEN), W2, Wout,
                bout.reshape(1, N_CLASSES))


# speedup vs baseline: 1.1568x; 1.1568x over previous
import functools

import jax
import jax.numpy as jnp
from jax import lax
from jax.experimental import pallas as pl
from jax.experimental.pallas import tpu as pltpu
from jax.experimental.pallas import tpu_sc as plsc

N_NODES = 10000
ACC_ROWS = 10008
D_FEAT = 128
HIDDEN = 256
N_CLASSES = 40
N_EDGES = 320000

NC = 2
NS = 16
NW = NC * NS
LANES = 16

E_PER_W = N_EDGES // NW
CHUNK = 128
SUB = 8
SUP_A = 10
SUP_B = 20
HALF = HIDDEN // 2
OWN = 624


@functools.partial(
    pl.kernel,
    out_type=jax.ShapeDtypeStruct((NW, N_NODES), jnp.float32),
    mesh=plsc.VectorSubcoreMesh(core_axis_name="c", subcore_axis_name="s"),
    scratch_types=[
        pltpu.VMEM((E_PER_W,), jnp.int32),
        pltpu.VMEM((N_NODES,), jnp.float32),
    ],
    compiler_params=pltpu.CompilerParams(needs_layout_passes=False),
)
def _deg_kernel(dst_hbm, out_hbm, idx_v, deg_v):
    c = lax.axis_index("c")
    s = lax.axis_index("s")
    wid = s * NC + c
    pltpu.sync_copy(dst_hbm.at[pl.ds(wid * E_PER_W, E_PER_W)], idx_v)

    zeros = jnp.zeros((LANES,), jnp.float32)

    @pl.loop(0, N_NODES // LANES)
    def _(i):
        deg_v[pl.ds(i * LANES, LANES)] = zeros

    ones = jnp.ones((LANES,), jnp.float32)

    @pl.loop(0, E_PER_W // LANES)
    def _(i):
        idx = idx_v[pl.ds(i * LANES, LANES)]
        plsc.addupdate_scatter(deg_v, [idx], ones)

    pltpu.sync_copy(deg_v, out_hbm.at[wid])


def _make_scatter_kernel(sup, shared_table=False):
    @functools.partial(
        pl.kernel,
        out_type=jax.ShapeDtypeStruct((NC, N_NODES, HALF), jnp.float32),
        mesh=plsc.VectorSubcoreMesh(core_axis_name="c", subcore_axis_name="s"),
        scratch_types=[
            pltpu.VMEM((SUB, CHUNK), jnp.int32),
            pltpu.VMEM((SUB, CHUNK), jnp.int32),
            pltpu.VMEM((SUB, CHUNK), jnp.int32),
            pltpu.VMEM((SUB, CHUNK), jnp.int32),
            pltpu.VMEM((CHUNK, HALF), jnp.float32),
            pltpu.VMEM((CHUNK, HALF), jnp.float32),
            pltpu.VMEM_SHARED((ACC_ROWS, HALF), jnp.float32),
            pltpu.SemaphoreType.DMA,
            pltpu.SemaphoreType.DMA,
            pltpu.SemaphoreType.DMA,
            pltpu.SemaphoreType.DMA,
            pltpu.SemaphoreType.DMA,
        ],
    )
    def _scatter_kernel(g_hbm, src_hbm, dst_hbm, out_hbm,
                        isrc_a, isrc_b, idst_a, idst_b, buf_a, buf_b, acc,
                        sem_ga, sem_gb, sem_sa, sem_sb, sem_i):
        c = lax.axis_index("c")
        s = lax.axis_index("s")
        isrc = (isrc_a, isrc_b)
        idst = (idst_a, idst_b)
        bufs = (buf_a, buf_b)
        gsems = (sem_ga, sem_gb)
        ssems = (sem_sa, sem_sb)

        pltpu.sync_copy(src_hbm.at[c, s, pl.ds(0, SUB)], isrc_a)
        pltpu.sync_copy(dst_hbm.at[c, s, pl.ds(0, SUB)], idst_a)

        zeros = jnp.zeros((LANES,), jnp.float32)

        @pl.loop(0, CHUNK)
        def _(i):
            @pl.loop(0, HALF // LANES)
            def _(k):
                buf_a[i, pl.ds(k * LANES, LANES)] = zeros

        zrows = 104
        zbase = pl.multiple_of(s * OWN, 8)

        @pl.loop(0, OWN // zrows)
        def _(i):
            pltpu.sync_copy(buf_a.at[pl.ds(0, zrows)],
                            acc.at[pl.ds(zbase + i * zrows, zrows)])

        @pl.when(s == NS - 1)
        def _():
            pltpu.sync_copy(buf_a.at[pl.ds(0, 16)],
                            acc.at[pl.ds(NS * OWN, 16)])

        plsc.subcore_barrier()

        g_c = g_hbm if shared_table else g_hbm.at[c]
        pltpu.async_copy(g_c.at[isrc_a.at[0]], buf_a, sem_ga)

        def super_step(m, sel):
            isrc_cur, idst_cur = isrc[sel], idst[sel]
            isrc_nxt, idst_nxt = isrc[1 - sel], idst[1 - sel]

            @pl.when(m < sup - 1)
            def _():
                nxt = pl.ds((m + 1) * SUB, SUB)
                pltpu.async_copy(src_hbm.at[c, s, nxt], isrc_nxt, sem_i)
                pltpu.async_copy(dst_hbm.at[c, s, nxt], idst_nxt, sem_i)

            for k in range(SUB):
                x = k % 2
                buf, other = bufs[x], bufs[1 - x]
                pltpu.make_async_copy(g_c.at[isrc_cur.at[k]], buf,
                                      gsems[x]).wait()
                pltpu.async_copy(buf, acc.at[idst_cur.at[k]], ssems[x],
                                 add=True)

                def wait_prev_scatter():
                    pltpu.make_async_copy(other, acc.at[idst_cur.at[k]],
                                          ssems[1 - x]).wait()

                if k > 0:
                    wait_prev_scatter()
                else:
                    @pl.when(m > 0)
                    def _():
                        wait_prev_scatter()

                if k < SUB - 1:
                    pltpu.async_copy(g_c.at[isrc_cur.at[k + 1]], other,
                                     gsems[1 - x])
                else:
                    @pl.when(m < sup - 1)
                    def _():
                        nxt = pl.ds((m + 1) * SUB, SUB)
                        pltpu.make_async_copy(
                            src_hbm.at[c, s, nxt], isrc_nxt, sem_i).wait()
                        pltpu.make_async_copy(
                            dst_hbm.at[c, s, nxt], idst_nxt, sem_i).wait()
                        pltpu.async_copy(g_c.at[isrc_nxt.at[0]], other,
                                         gsems[1 - x])

        @pl.loop(0, sup // 2)
        def _(mm):
            super_step(mm * 2, 0)
            super_step(mm * 2 + 1, 1)

        pltpu.make_async_copy(bufs[1], acc.at[idst_b.at[SUB - 1]],
                              ssems[1]).wait()

        plsc.subcore_barrier()
        wbase = pl.multiple_of(s * OWN, 8)
        pltpu.sync_copy(acc.at[pl.ds(wbase, OWN)],
                        out_hbm.at[c, pl.ds(wbase, OWN)])

        @pl.when(s == NS - 1)
        def _():
            pltpu.sync_copy(acc.at[pl.ds(NS * OWN, 16)],
                            out_hbm.at[c, pl.ds(NS * OWN, 16)])

    return _scatter_kernel


_scatter_a = _make_scatter_kernel(SUP_A)
_scatter_b = _make_scatter_kernel(SUP_B)


_RB = 2000
_GRID = N_NODES // _RB


def _tc1_body(x_ref, dp_ref, y_ref, dinv_ref):
    deg = jnp.sum(dp_ref[...], axis=1, keepdims=True) + 1.0
    dinv = lax.rsqrt(deg)
    y_ref[...] = x_ref[...] * dinv
    dinv_ref[...] = dinv


def _tc1(x, dp):
    return pl.pallas_call(
        _tc1_body,
        grid=(_GRID,),
        in_specs=[
            pl.BlockSpec((_RB, D_FEAT), lambda i: (i, 0)),
            pl.BlockSpec((_RB, NW), lambda i: (i, 0)),
        ],
        out_specs=[
            pl.BlockSpec((_RB, D_FEAT), lambda i: (i, 0)),
            pl.BlockSpec((_RB, 1), lambda i: (i, 0)),
        ],
        out_shape=[
            jax.ShapeDtypeStruct((N_NODES, D_FEAT), jnp.float32),
            jax.ShapeDtypeStruct((N_NODES, 1), jnp.float32),
        ],
    )(x, dp)


def _tc2_body(s_ref, y_ref, dinv_ref, b_ref, w_ref, y2_ref):
    dinv = dinv_ref[...]
    z = s_ref[0] + s_ref[1] + y_ref[...]
    h = jnp.maximum(
        dinv * jnp.dot(z, w_ref[...], preferred_element_type=jnp.float32)
        + b_ref[...], 0.0)
    y2 = h * dinv
    y2_ref[0] = y2[:, :HALF]
    y2_ref[1] = y2[:, HALF:]


def _tc2(Sy1, y1, dinv, b1, W1):
    return pl.pallas_call(
        _tc2_body,
        grid=(_GRID,),
        in_specs=[
            pl.BlockSpec((NC, _RB, D_FEAT), lambda i: (0, i, 0)),
            pl.BlockSpec((_RB, D_FEAT), lambda i: (i, 0)),
            pl.BlockSpec((_RB, 1), lambda i: (i, 0)),
            pl.BlockSpec((1, HIDDEN), lambda i: (0, 0)),
            pl.BlockSpec((D_FEAT, HIDDEN), lambda i: (0, 0)),
        ],
        out_specs=pl.BlockSpec((NC, _RB, HALF), lambda i: (0, i, 0)),
        out_shape=jax.ShapeDtypeStruct((NC, N_NODES, HALF), jnp.float32),
    )(Sy1, y1, dinv, b1, W1)


def _tc3_body(s_ref, y2_ref, dinv_ref, b_ref, w_ref, wo_ref, bo_ref,
              out_ref):
    dinv = dinv_ref[...]
    w = w_ref[...]
    z0 = s_ref[0] + y2_ref[0]
    z1 = s_ref[1] + y2_ref[1]
    h = jnp.maximum(
        dinv * (jnp.dot(z0, w[:HALF], preferred_element_type=jnp.float32)
                + jnp.dot(z1, w[HALF:], preferred_element_type=jnp.float32))
        + b_ref[...], 0.0)
    logits = (jnp.dot(h, wo_ref[...], preferred_element_type=jnp.float32)
              + bo_ref[...])
    m = jnp.max(logits, axis=1, keepdims=True)
    e = jnp.exp(logits - m)
    out_ref[...] = e / jnp.sum(e, axis=1, keepdims=True)


def _tc3(S2, y2, dinv, b2, W2, Wout, bout):
    return pl.pallas_call(
        _tc3_body,
        grid=(_GRID,),
        in_specs=[
            pl.BlockSpec((NC, _RB, HALF), lambda i: (0, i, 0)),
            pl.BlockSpec((NC, _RB, HALF), lambda i: (0, i, 0)),
            pl.BlockSpec((_RB, 1), lambda i: (i, 0)),
            pl.BlockSpec((1, HIDDEN), lambda i: (0, 0)),
            pl.BlockSpec((HIDDEN, HIDDEN), lambda i: (0, 0)),
            pl.BlockSpec((HIDDEN, N_CLASSES), lambda i: (0, 0)),
            pl.BlockSpec((1, N_CLASSES), lambda i: (0, 0)),
        ],
        out_specs=pl.BlockSpec((_RB, N_CLASSES), lambda i: (i, 0)),
        out_shape=jax.ShapeDtypeStruct((N_NODES, N_CLASSES), jnp.float32),
    )(S2, y2, dinv, b2, W2, Wout, bout)


def _pad_edges_split(idx, fill):
    n_real = N_EDGES // (NC * NS)
    n_pad = SUP_A * SUB * CHUNK
    idx3 = idx.reshape(NC, NS, n_real)
    pad = jnp.full((NC, NS, n_pad - n_real), fill, jnp.int32)
    return jnp.concatenate([idx3, pad], axis=2).reshape(
        NC, NS, SUP_A * SUB, CHUNK)


def _pad_edges_full(idx, fill):
    n_real = N_EDGES // NS
    n_pad = SUP_B * SUB * CHUNK
    idx2 = idx.reshape(NS, n_real)
    pad = jnp.full((NS, n_pad - n_real), fill, jnp.int32)
    arr = jnp.concatenate([idx2, pad], axis=1).reshape(
        NS, SUP_B * SUB, CHUNK)
    return jnp.broadcast_to(arr[None], (NC,) + arr.shape)


@jax.jit
def kernel(x, edge_index, W1, b1, W2, b2, Wout, bout):
    src = edge_index[0].astype(jnp.int32)
    dst = edge_index[1].astype(jnp.int32)

    deg_parts = _deg_kernel(dst)
    dp = deg_parts.T

    srcA = _pad_edges_split(src, 0)
    dstA = _pad_edges_split(dst, N_NODES)
    srcB = _pad_edges_full(src, 0)
    dstB = _pad_edges_full(dst, N_NODES)

    y1, dinv = _tc1(x, dp)
    y1c = jnp.broadcast_to(y1[None], (NC, N_NODES, D_FEAT))
    Sy1 = _scatter_a(y1c, srcA, dstA)
    y2 = _tc2(Sy1, y1, dinv, b1.reshape(1, HIDDEN), W1)
    S2 = _scatter_b(y2, srcB, dstB)
    return _tc3(S2, y2, dinv, b2.reshape(1, HIDDEN), W2, Wout,
                bout.reshape(1, N_CLASSES))
